# Initial kernel scaffold; baseline (speedup 1.0000x reference)
import jax
import jax.numpy as jnp
from jax.experimental import pallas as pl


def kernel(inputs, input_table, position_table):
    raise NotImplementedError("write your pallas kernel here")


# speedup vs baseline: 1.2749x; 1.2749x over previous
import jax
import jax.numpy as jnp
from jax import lax
from jax.experimental import pallas as pl
from jax.experimental.pallas import tpu as pltpu
from jax.experimental.pallas import tpu_sc as plsc

_NC = 2
_NS = 16
_NW = _NC * _NS
_LANES = 16


def _embed_kernel(idx_hbm, tok_hbm, pos_hbm, out_hbm, idx_v, rows_v, pos_v, sem):
    n, embed = out_hbm.shape
    seqlen = pos_hbm.shape[0]
    chunk = n // _NW
    wid = lax.axis_index("s") * _NC + lax.axis_index("c")
    base = wid * chunk
    pos_base = lax.rem(base, seqlen)

    pltpu.sync_copy(idx_hbm.at[pl.ds(base, chunk)], idx_v)
    gather = pltpu.async_copy(tok_hbm.at[idx_v], rows_v, sem)
    pltpu.sync_copy(pos_hbm.at[pl.ds(pos_base, chunk)], pos_v)
    gather.wait()

    @pl.loop(0, chunk)
    def _row(i):
        @pl.loop(0, embed, step=_LANES)
        def _lane(j):
            slc = (pl.ds(i, 1), pl.ds(j, _LANES))
            rows_v.at[*slc][...] = rows_v.at[*slc][...] + pos_v.at[*slc][...]

    pltpu.sync_copy(rows_v, out_hbm.at[pl.ds(base, chunk)])


def kernel(inputs, input_table, position_table):
    batch, seqlen = inputs.shape
    vocab, embed = input_table.shape
    n = batch * seqlen
    chunk = n // _NW
    idx_flat = inputs.reshape(n).astype(jnp.int32)

    mesh = plsc.VectorSubcoreMesh(core_axis_name="c", subcore_axis_name="s")
    run = pl.kernel(
        _embed_kernel,
        out_type=jax.ShapeDtypeStruct((n, embed), jnp.float32),
        mesh=mesh,
        scratch_types=[
            pltpu.VMEM((chunk,), jnp.int32),
            pltpu.VMEM((chunk, embed), jnp.float32),
            pltpu.VMEM((chunk, embed), jnp.float32),
            pltpu.SemaphoreType.DMA,
        ],
    )
    out = run(idx_flat, input_table, position_table)
    return out.reshape(batch, seqlen, embed)


# speedup vs baseline: 1.3585x; 1.0656x over previous
import jax
import jax.numpy as jnp
from jax import lax
from jax.experimental import pallas as pl
from jax.experimental.pallas import tpu as pltpu
from jax.experimental.pallas import tpu_sc as plsc

_NC = 2
_NS = 16
_NW = _NC * _NS
_LANES = 16


def _embed_kernel(idx_hbm, tok_hbm, pos_hbm, out_hbm, idx_v, rows_v, sem):
    n, embed = out_hbm.shape
    seqlen = pos_hbm.shape[0]
    chunk = n // _NW
    wid = lax.axis_index("s") * _NC + lax.axis_index("c")
    base = wid * chunk
    pos_base = lax.rem(base, seqlen)

    pltpu.sync_copy(idx_hbm.at[pl.ds(base, chunk)], idx_v)
    pltpu.sync_copy(pos_hbm.at[pl.ds(pos_base, chunk)], rows_v)
    pltpu.async_copy(tok_hbm.at[idx_v], rows_v, sem, add=True).wait()
    pltpu.sync_copy(rows_v, out_hbm.at[pl.ds(base, chunk)])


def kernel(inputs, input_table, position_table):
    batch, seqlen = inputs.shape
    vocab, embed = input_table.shape
    n = batch * seqlen
    chunk = n // _NW
    idx_flat = inputs.reshape(n).astype(jnp.int32)

    mesh = plsc.VectorSubcoreMesh(core_axis_name="c", subcore_axis_name="s")
    run = pl.kernel(
        _embed_kernel,
        out_type=jax.ShapeDtypeStruct((n, embed), jnp.float32),
        mesh=mesh,
        scratch_types=[
            pltpu.VMEM((chunk,), jnp.int32),
            pltpu.VMEM((chunk, embed), jnp.float32),
            pltpu.SemaphoreType.DMA,
        ],
    )
    out = run(idx_flat, input_table, position_table)
    return out.reshape(batch, seqlen, embed)


# speedup vs baseline: 1.3765x; 1.0133x over previous
import jax
import jax.numpy as jnp
from jax import lax
from jax.experimental import pallas as pl
from jax.experimental.pallas import tpu as pltpu
from jax.experimental.pallas import tpu_sc as plsc

_NC = 2
_NS = 16
_NW = _NC * _NS
_SUB = 4


def _embed_kernel(idx_hbm, tok_hbm, pos_hbm, out_hbm, idx_v, rows_v, isem,
                  psem, gsem, osem):
    n, embed = out_hbm.shape
    seqlen = pos_hbm.shape[0]
    chunk = n // _NW
    sub = chunk // _SUB
    wid = lax.axis_index("s") * _NC + lax.axis_index("c")
    base = wid * chunk
    pos_base = lax.rem(base, seqlen)

    idx_cp = pltpu.async_copy(idx_hbm.at[pl.ds(wid * _SUB, _SUB)], idx_v, isem)
    pre = [
        pltpu.async_copy(
            pos_hbm.at[pl.ds(pos_base + j * sub, sub)],
            rows_v.at[pl.ds(j * sub, sub)],
            psem.at[j],
        )
        for j in range(_SUB)
    ]
    idx_cp.wait()
    gat = []
    for j in range(_SUB):
        pre[j].wait()
        gat.append(
            pltpu.async_copy(
                tok_hbm.at[idx_v.at[j]],
                rows_v.at[pl.ds(j * sub, sub)],
                gsem.at[j],
                add=True,
            )
        )
    out = []
    for j in range(_SUB):
        gat[j].wait()
        out.append(
            pltpu.async_copy(
                rows_v.at[pl.ds(j * sub, sub)],
                out_hbm.at[pl.ds(base + j * sub, sub)],
                osem.at[j],
            )
        )
    for j in range(_SUB):
        out[j].wait()


def kernel(inputs, input_table, position_table):
    batch, seqlen = inputs.shape
    vocab, embed = input_table.shape
    n = batch * seqlen
    chunk = n // _NW
    sub = chunk // _SUB
    idx_2d = inputs.reshape(_NW * _SUB, sub).astype(jnp.int32)

    mesh = plsc.VectorSubcoreMesh(core_axis_name="c", subcore_axis_name="s")
    run = pl.kernel(
        _embed_kernel,
        out_type=jax.ShapeDtypeStruct((n, embed), jnp.float32),
        mesh=mesh,
        scratch_types=[
            pltpu.VMEM((_SUB, sub), jnp.int32),
            pltpu.VMEM((chunk, embed), jnp.float32),
            pltpu.SemaphoreType.DMA,
            pltpu.SemaphoreType.DMA((_SUB,)),
            pltpu.SemaphoreType.DMA((_SUB,)),
            pltpu.SemaphoreType.DMA((_SUB,)),
        ],
    )
    out = run(idx_2d, input_table, position_table)
    return out.reshape(batch, seqlen, embed)


# speedup vs baseline: 1.3877x; 1.0082x over previous
import jax
import jax.numpy as jnp
from jax import lax
from jax.experimental import pallas as pl
from jax.experimental.pallas import tpu as pltpu
from jax.experimental.pallas import tpu_sc as plsc

_NC = 2
_NS = 16
_NW = _NC * _NS
_SUB = 4


def _embed_kernel(idx_hbm, tok_hbm, pos_hbm, out_hbm, idx_v, rows_v, isem,
                  psem, gsem, osem):
    n, embed = out_hbm.shape
    batch, seqlen = idx_hbm.shape
    chunk = n // _NW
    sub = chunk // _SUB
    wid = lax.axis_index("s") * _NC + lax.axis_index("c")
    base = wid * chunk
    b = base // seqlen
    pos_base = lax.rem(base, seqlen)

    idx_cp = pltpu.async_copy(idx_hbm.at[b, pl.ds(pos_base, chunk)], idx_v, isem)
    pre = [
        pltpu.async_copy(
            pos_hbm.at[pl.ds(pos_base + j * sub, sub)],
            rows_v.at[pl.ds(j * sub, sub)],
            psem.at[j],
        )
        for j in range(_SUB)
    ]
    idx_cp.wait()
    gat = []
    for j in range(_SUB):
        pre[j].wait()
        gat.append(
            pltpu.async_copy(
                tok_hbm.at[idx_v.at[pl.ds(j * sub, sub)]],
                rows_v.at[pl.ds(j * sub, sub)],
                gsem.at[j],
                add=True,
            )
        )
    out = []
    for j in range(_SUB):
        gat[j].wait()
        out.append(
            pltpu.async_copy(
                rows_v.at[pl.ds(j * sub, sub)],
                out_hbm.at[pl.ds(base + j * sub, sub)],
                osem.at[j],
            )
        )
    for j in range(_SUB):
        out[j].wait()


def kernel(inputs, input_table, position_table):
    batch, seqlen = inputs.shape
    vocab, embed = input_table.shape
    n = batch * seqlen
    chunk = n // _NW

    mesh = plsc.VectorSubcoreMesh(core_axis_name="c", subcore_axis_name="s")
    run = pl.kernel(
        _embed_kernel,
        out_type=jax.ShapeDtypeStruct((n, embed), jnp.float32),
        mesh=mesh,
        scratch_types=[
            pltpu.VMEM((chunk,), jnp.int32),
            pltpu.VMEM((chunk, embed), jnp.float32),
            pltpu.SemaphoreType.DMA,
            pltpu.SemaphoreType.DMA((_SUB,)),
            pltpu.SemaphoreType.DMA((_SUB,)),
            pltpu.SemaphoreType.DMA((_SUB,)),
        ],
    )
    out = run(inputs.astype(jnp.int32), input_table, position_table)
    return out.reshape(batch, seqlen, embed)
